# Initial kernel scaffold; baseline (speedup 1.0000x reference)
import jax
import jax.numpy as jnp
from jax.experimental import pallas as pl


def kernel(entity_emb, relation_emb, item_bias, entity_pairs, entity2edges, edge2entities, edge2relation):
    raise NotImplementedError("write your pallas kernel here")


# speedup vs baseline: 3.5273x; 3.5273x over previous
import jax
import jax.numpy as jnp
from jax import lax
from jax.experimental import pallas as pl
from jax.experimental.pallas import tpu as pltpu
from jax.experimental.pallas import tpu_sc as plsc

DIM = 128
NS = 32
BS = 4096
NSLOTS = 2 * BS
NW = 32
SLOTS_PER_W = NSLOTS // NW
PAIRS_PER_W = BS // NW
CHUNK_SLOTS = 4
NCHUNK = SLOTS_PER_W // CHUNK_SLOTS
VPR = DIM // 16
PACKW = 16


def _sc_body(ent_emb, rel_emb, pair_flat, e2edges, epack,
             out, pidx, edgebuf, epk, eflat, eeflat, rflat,
             entbuf, relbuf, selfbuf, scorebuf, sem0, sem1, sem2, sem3):
    wid = lax.axis_index("s") * 2 + lax.axis_index("c")
    base = wid * SLOTS_PER_W

    pltpu.sync_copy(pair_flat.at[pl.ds(base, SLOTS_PER_W)], pidx)

    cp_e0 = pltpu.async_copy(e2edges.at[pidx.at[pl.ds(0, 128)]],
                             edgebuf.at[pl.ds(0, 128)], sem0)
    cp_e1 = pltpu.async_copy(e2edges.at[pidx.at[pl.ds(128, 128)]],
                             edgebuf.at[pl.ds(128, 128)], sem0)
    cp_s0 = pltpu.async_copy(ent_emb.at[pidx.at[pl.ds(0, 128)]],
                             selfbuf.at[pl.ds(0, 128)], sem1)
    cp_s1 = pltpu.async_copy(ent_emb.at[pidx.at[pl.ds(128, 128)]],
                             selfbuf.at[pl.ds(128, 128)], sem1)
    cp_e0.wait()
    cp_e1.wait()
    cp_s0.wait()
    cp_s1.wait()

    iota = jnp.arange(16, dtype=jnp.int32)
    zeros16 = jnp.zeros((16,), jnp.int32)
    twos16 = zeros16 + 2
    inv64 = jnp.float32(1.0 / 64.0)

    def chunk_body(c, carry):
        for k in range(8):
            row = c * CHUNK_SLOTS + k // 2
            eflat[pl.ds(k * 16, 16)] = edgebuf[row, pl.ds((k % 2) * 16, 16)]
        cpa = pltpu.async_copy(epack.at[eflat], epk, sem2)
        cpa.wait()
        for k in range(16):
            pos = k * 16 + iota
            eeflat[pl.ds(k * 16, 16)] = plsc.load_gather(
                epk, [pos >> 1, pos & 1])
        for k in range(8):
            rflat[pl.ds(k * 16, 16)] = plsc.load_gather(
                epk, [k * 16 + iota, twos16])
        g0 = pltpu.async_copy(ent_emb.at[eeflat.at[pl.ds(0, 128)]],
                              entbuf.at[pl.ds(0, 128)], sem0)
        g1 = pltpu.async_copy(ent_emb.at[eeflat.at[pl.ds(128, 128)]],
                              entbuf.at[pl.ds(128, 128)], sem1)
        g2 = pltpu.async_copy(rel_emb.at[rflat], relbuf, sem3)
        g0.wait()
        g1.wait()
        g2.wait()

        for s in range(CHUNK_SLOTS):
            def red_body(j, acc):
                er = s * 64 + 2 * j
                rrow = s * NS + j
                out_acc = []
                for v in range(VPR):
                    sl = pl.ds(v * 16, 16)
                    a = entbuf[er, sl] + entbuf[er + 1, sl]
                    b = relbuf[rrow, sl]
                    out_acc.append(acc[v] + a + b + b)
                return tuple(out_acc)

            zero = jnp.zeros((16,), jnp.float32)
            acc = lax.fori_loop(0, NS, red_body, (zero,) * VPR)
            slot = c * CHUNK_SLOTS + s
            for v in range(VPR):
                sl = pl.ds(v * 16, 16)
                selfbuf[slot, sl] = selfbuf[slot, sl] + acc[v] * inv64
        return carry

    lax.fori_loop(0, NCHUNK, chunk_body, 0)

    for g in range(PAIRS_PER_W // 16):
        rows0 = 2 * (g * 16 + iota)
        rows1 = rows0 + 1

        def dot_body(d, sc):
            cols = zeros16 + d
            a = plsc.load_gather(selfbuf, [rows0, cols])
            b = plsc.load_gather(selfbuf, [rows1, cols])
            return sc + a * b

        sc = lax.fori_loop(0, DIM, dot_body, jnp.zeros((16,), jnp.float32))
        scorebuf[pl.ds(g * 16, 16)] = sc

    pltpu.sync_copy(scorebuf, out.at[pl.ds(wid * PAIRS_PER_W, PAIRS_PER_W)])


@jax.jit
def _run(ent_emb, rel_emb, pair_flat, e2edges, epack):
    mesh = plsc.VectorSubcoreMesh(core_axis_name="c", subcore_axis_name="s")
    f = pl.kernel(
        _sc_body,
        out_type=jax.ShapeDtypeStruct((BS,), jnp.float32),
        mesh=mesh,
        scratch_types=[
            pltpu.VMEM((SLOTS_PER_W,), jnp.int32),
            pltpu.VMEM((SLOTS_PER_W, NS), jnp.int32),
            pltpu.VMEM((128, PACKW), jnp.int32),
            pltpu.VMEM((128,), jnp.int32),
            pltpu.VMEM((256,), jnp.int32),
            pltpu.VMEM((128,), jnp.int32),
            pltpu.VMEM((256, DIM), jnp.float32),
            pltpu.VMEM((128, DIM), jnp.float32),
            pltpu.VMEM((SLOTS_PER_W, DIM), jnp.float32),
            pltpu.VMEM((PAIRS_PER_W,), jnp.float32),
            pltpu.SemaphoreType.DMA,
            pltpu.SemaphoreType.DMA,
            pltpu.SemaphoreType.DMA,
            pltpu.SemaphoreType.DMA,
        ],
        compiler_params=pltpu.CompilerParams(
            needs_layout_passes=False, use_tc_tiling_on_sc=False),
    )
    return f(ent_emb, rel_emb, pair_flat, e2edges, epack)


def kernel(entity_emb, relation_emb, item_bias, entity_pairs, entity2edges,
           edge2entities, edge2relation):
    pair_flat = entity_pairs.reshape(NSLOTS).astype(jnp.int32)
    n_edges = edge2relation.shape[0]
    epack = jnp.concatenate(
        [edge2entities, edge2relation.reshape(-1, 1),
         jnp.zeros((n_edges, PACKW - 3), jnp.int32)], axis=1)
    score = _run(entity_emb, relation_emb, pair_flat, entity2edges, epack)
    return score + jnp.take(item_bias, entity_pairs[:, 1], axis=0)
